# baseline (speedup 1.0000x reference)
import functools

import jax
import jax.numpy as jnp
from jax import lax
from jax.experimental import pallas as pl
from jax.experimental.pallas import tpu as pltpu
from jax.experimental.pallas import tpu_sc as plsc

N = 10000
D = 128
E = 320000

NC = 2
NS = 16
NW = NC * NS
CHUNK = 120
CPW = 84
EPAD = NW * CPW * CHUNK
NPAD = 10112
RPT = NPAD // NS
R = 3
I = 4
UNR = 12


def _make_sc_agg():
    mesh = plsc.VectorSubcoreMesh(core_axis_name="c", subcore_axis_name="s")

    @functools.partial(
        pl.kernel, mesh=mesh,
        out_type=jax.ShapeDtypeStruct((NC, NPAD, D), jnp.float32),
        scratch_types=[
            pltpu.VMEM((I, CHUNK), jnp.int32),
            pltpu.VMEM((I, CHUNK), jnp.int32),
            pltpu.VMEM((R, CHUNK, D), jnp.float32),
            pltpu.VMEM_SHARED((NPAD, D), jnp.float32),
        ] + [pltpu.SemaphoreType.DMA] * (2 * R + I),
    )
    def sc_agg(x_hbm, src_hbm, dst_hbm, z_hbm, out_hbm,
               src_v, dst_v, rows_v, agg_sh, *sems):
        cid = lax.axis_index("c")
        sid = lax.axis_index("s")
        wid = sid * NC + cid
        sg = sems[:R]
        ss = sems[R:2 * R]
        si = sems[2 * R:]
        pltpu.sync_copy(z_hbm, agg_sh.at[pl.ds(sid * RPT, RPT)])
        plsc.subcore_barrier()

        def _idx_load(j, bi):
            pltpu.async_copy(src_hbm.at[wid, j], src_v.at[bi], si[bi])
            pltpu.async_copy(dst_hbm.at[wid, j], dst_v.at[bi], si[bi])

        def _idx_wait(bi):
            pltpu.make_async_copy(src_hbm.at[wid, 0], src_v.at[bi],
                                  si[bi]).wait()
            pltpu.make_async_copy(dst_hbm.at[wid, 0], dst_v.at[bi],
                                  si[bi]).wait()

        SPL = 64

        def _gather(bi, br):
            pltpu.async_copy(x_hbm.at[src_v.at[bi, pl.ds(0, SPL)]],
                             rows_v.at[br, pl.ds(0, SPL)], sg[br])
            pltpu.async_copy(x_hbm.at[src_v.at[bi, pl.ds(SPL, CHUNK - SPL)]],
                             rows_v.at[br, pl.ds(SPL, CHUNK - SPL)], sg[br])

        def _gather_wait(bi, br):
            pltpu.make_async_copy(x_hbm.at[src_v.at[bi, pl.ds(0, SPL)]],
                                  rows_v.at[br, pl.ds(0, SPL)],
                                  sg[br]).wait()
            pltpu.make_async_copy(x_hbm.at[src_v.at[bi, pl.ds(SPL, CHUNK - SPL)]],
                                  rows_v.at[br, pl.ds(SPL, CHUNK - SPL)],
                                  sg[br]).wait()

        def _scatter(bi, br):
            pltpu.async_copy(rows_v.at[br], agg_sh.at[dst_v.at[bi]], ss[br],
                             add=True)

        def _scatter_wait(bi, br):
            pltpu.make_async_copy(rows_v.at[br], agg_sh.at[dst_v.at[bi]],
                                  ss[br]).wait()

        for p in range(R):
            _idx_load(p, p)
        for p in range(R - 1):
            _idx_wait(p)
            _gather(p, p)

        def body(i, carry):
            j = i * UNR
            for u in range(UNR):
                k = j + u
                br = u % R
                bi = u % I
                _gather_wait(bi, br)
                _scatter(bi, br)
                @pl.when(k >= 1)
                def _():
                    _scatter_wait((u + I - 1) % I, (u + R - 1) % R)
                @pl.when(k + R < CPW)
                def _():
                    _idx_load(k + R, (u + R) % I)
                @pl.when(k + R - 1 < CPW)
                def _():
                    _idx_wait((u + R - 1) % I)
                    _gather((u + R - 1) % I, (u + R - 1) % R)
            return carry

        lax.fori_loop(0, CPW // UNR, body, 0)
        _scatter_wait((CPW - 1) % I, (CPW - 1) % R)
        plsc.subcore_barrier()
        pltpu.sync_copy(agg_sh.at[pl.ds(sid * RPT, RPT)],
                        out_hbm.at[cid, pl.ds(sid * RPT, RPT)])

    return sc_agg


_sc_agg = _make_sc_agg()


def _proj_body(a_ref, w1_ref, b1_ref, w2_ref, b2_ref, o_ref):
    t = jnp.dot(a_ref[0] + a_ref[1], w1_ref[...],
                preferred_element_type=jnp.float32)
    h = jnp.maximum(t + b1_ref[...], 0.0)
    o_ref[...] = jnp.sum(h * w2_ref[...], axis=1, keepdims=True) + b2_ref[0, 0]


def _proj(agg2, W1, b1r, w2r, b2r):
    BLK = 2528
    return pl.pallas_call(
        _proj_body,
        grid=(NPAD // BLK,),
        in_specs=[pl.BlockSpec((NC, BLK, D), lambda i: (0, i, 0)),
                  pl.BlockSpec((D, D), lambda i: (0, 0)),
                  pl.BlockSpec((1, D), lambda i: (0, 0)),
                  pl.BlockSpec((1, D), lambda i: (0, 0)),
                  pl.BlockSpec((1, 1), lambda i: (0, 0))],
        out_specs=pl.BlockSpec((BLK, 1), lambda i: (i, 0)),
        out_shape=jax.ShapeDtypeStruct((NPAD, 1), jnp.float32),
    )(agg2, W1, b1r, w2r, b2r)


def kernel(x, edge_index, W1, b1, W2, b2):
    src = edge_index[0].astype(jnp.int32)
    dst = edge_index[1].astype(jnp.int32)
    pad = EPAD - E
    pidx = jnp.arange(pad, dtype=jnp.int32)
    src3 = jnp.concatenate([src, pidx % N]).reshape(NW, CPW, CHUNK)
    dst3 = jnp.concatenate([dst, N + pidx % (NPAD - N)]).reshape(NW, CPW, CHUNK)
    zeros = jnp.zeros((RPT, D), jnp.float32)

    agg2 = _sc_agg(x, src3, dst3, zeros)
    res = _proj(agg2, W1, b1.reshape(1, D), W2.reshape(1, D), b2.reshape(1, 1))
    return res[:N, 0]


# speedup vs baseline: 1.0328x; 1.0328x over previous
import functools

import jax
import jax.numpy as jnp
from jax import lax
from jax.experimental import pallas as pl
from jax.experimental.pallas import tpu as pltpu
from jax.experimental.pallas import tpu_sc as plsc

N = 10000
D = 128
E = 320000

NC = 2
NS = 16
NW = NC * NS
CHUNK = 120
CPW = 84
EPAD = NW * CPW * CHUNK
NPAD = 10112
RPT = NPAD // NS
R = 3
I = 4
UNR = 12


def _make_sc_agg():
    mesh = plsc.VectorSubcoreMesh(core_axis_name="c", subcore_axis_name="s")

    @functools.partial(
        pl.kernel, mesh=mesh,
        out_type=jax.ShapeDtypeStruct((NC, NPAD, D), jnp.float32),
        scratch_types=[
            pltpu.VMEM((I, CHUNK), jnp.int32),
            pltpu.VMEM((I, CHUNK), jnp.int32),
            pltpu.VMEM((R, CHUNK, D), jnp.float32),
            pltpu.VMEM_SHARED((NPAD, D), jnp.float32),
        ] + [pltpu.SemaphoreType.DMA] * (2 * R + I),
    )
    def sc_agg(x_hbm, src_hbm, dst_hbm, out_hbm,
               src_v, dst_v, rows_v, agg_sh, *sems):
        cid = lax.axis_index("c")
        sid = lax.axis_index("s")
        wid = sid * NC + cid
        sg = sems[:R]
        ss = sems[R:2 * R]
        si = sems[2 * R:]
        zv = jnp.zeros((16,), jnp.float32)

        def zbody(r, c):
            for cc in range(8):
                rows_v[0, r, pl.ds(cc * 16, 16)] = zv
            return c

        lax.fori_loop(0, CHUNK, zbody, 0)
        for t in range(RPT // CHUNK):
            pltpu.sync_copy(rows_v.at[0],
                            agg_sh.at[pl.ds(sid * RPT + t * CHUNK, CHUNK)])
        REM = RPT - (RPT // CHUNK) * CHUNK
        pltpu.sync_copy(rows_v.at[0, pl.ds(0, REM)],
                        agg_sh.at[pl.ds(sid * RPT + RPT - REM, REM)])
        plsc.subcore_barrier()

        def _idx_load(j, bi):
            pltpu.async_copy(src_hbm.at[wid, j], src_v.at[bi], si[bi])
            pltpu.async_copy(dst_hbm.at[wid, j], dst_v.at[bi], si[bi])

        def _idx_wait(bi):
            pltpu.make_async_copy(src_hbm.at[wid, 0], src_v.at[bi],
                                  si[bi]).wait()
            pltpu.make_async_copy(dst_hbm.at[wid, 0], dst_v.at[bi],
                                  si[bi]).wait()

        def _gather(bi, br):
            pltpu.async_copy(x_hbm.at[src_v.at[bi]], rows_v.at[br], sg[br])

        def _gather_wait(bi, br):
            pltpu.make_async_copy(x_hbm.at[src_v.at[bi]], rows_v.at[br],
                                  sg[br]).wait()

        def _scatter(bi, br):
            pltpu.async_copy(rows_v.at[br], agg_sh.at[dst_v.at[bi]], ss[br],
                             add=True)

        def _scatter_wait(bi, br):
            pltpu.make_async_copy(rows_v.at[br], agg_sh.at[dst_v.at[bi]],
                                  ss[br]).wait()

        for p in range(R):
            _idx_load(p, p)
        for p in range(R - 1):
            _idx_wait(p)
            _gather(p, p)

        def body(i, carry):
            j = i * UNR
            for u in range(UNR):
                k = j + u
                br = u % R
                bi = u % I
                _gather_wait(bi, br)
                _scatter(bi, br)
                @pl.when(k >= 1)
                def _():
                    _scatter_wait((u + I - 1) % I, (u + R - 1) % R)
                @pl.when(k + R < CPW)
                def _():
                    _idx_load(k + R, (u + R) % I)
                @pl.when(k + R - 1 < CPW)
                def _():
                    _idx_wait((u + R - 1) % I)
                    _gather((u + R - 1) % I, (u + R - 1) % R)
            return carry

        lax.fori_loop(0, CPW // UNR, body, 0)
        _scatter_wait((CPW - 1) % I, (CPW - 1) % R)
        plsc.subcore_barrier()
        pltpu.sync_copy(agg_sh.at[pl.ds(sid * RPT, RPT)],
                        out_hbm.at[cid, pl.ds(sid * RPT, RPT)])

    return sc_agg


_sc_agg = _make_sc_agg()


def _proj_body(a_ref, w1_ref, b1_ref, w2_ref, b2_ref, o_ref):
    t = jnp.dot(a_ref[0] + a_ref[1], w1_ref[...],
                preferred_element_type=jnp.float32)
    h = jnp.maximum(t + b1_ref[...], 0.0)
    o_ref[...] = jnp.sum(h * w2_ref[...], axis=1, keepdims=True) + b2_ref[0, 0]


def _proj(agg2, W1, b1r, w2r, b2r):
    BLK = 2528
    return pl.pallas_call(
        _proj_body,
        grid=(NPAD // BLK,),
        in_specs=[pl.BlockSpec((NC, BLK, D), lambda i: (0, i, 0)),
                  pl.BlockSpec((D, D), lambda i: (0, 0)),
                  pl.BlockSpec((1, D), lambda i: (0, 0)),
                  pl.BlockSpec((1, D), lambda i: (0, 0)),
                  pl.BlockSpec((1, 1), lambda i: (0, 0))],
        out_specs=pl.BlockSpec((BLK, 1), lambda i: (i, 0)),
        out_shape=jax.ShapeDtypeStruct((NPAD, 1), jnp.float32),
    )(agg2, W1, b1r, w2r, b2r)


def kernel(x, edge_index, W1, b1, W2, b2):
    src = edge_index[0].astype(jnp.int32)
    dst = edge_index[1].astype(jnp.int32)
    pad = EPAD - E
    pidx = jnp.arange(pad, dtype=jnp.int32)
    src3 = jnp.concatenate([src, pidx % N]).reshape(NW, CPW, CHUNK)
    dst3 = jnp.concatenate([dst, N + pidx % (NPAD - N)]).reshape(NW, CPW, CHUNK)
    agg2 = _sc_agg(x, src3, dst3)
    res = _proj(agg2, W1, b1.reshape(1, D), W2.reshape(1, D), b2.reshape(1, 1))
    return res[:N, 0]


# speedup vs baseline: 1.0712x; 1.0372x over previous
import functools

import jax
import jax.numpy as jnp
from jax import lax
from jax.experimental import pallas as pl
from jax.experimental.pallas import tpu as pltpu
from jax.experimental.pallas import tpu_sc as plsc

N = 10000
D = 128
E = 320000

NC = 2
NS = 16
NW = NC * NS
CHUNK = 120
CPW = 84
EPAD = NW * CPW * CHUNK
NPAD = 10112
RPT = NPAD // NS
R = 3
I = 4
UNR = 12


def _make_sc_agg():
    mesh = plsc.VectorSubcoreMesh(core_axis_name="c", subcore_axis_name="s")

    @functools.partial(
        pl.kernel, mesh=mesh,
        out_type=jax.ShapeDtypeStruct((NC, NPAD, D), jnp.float32),
        scratch_types=[
            pltpu.VMEM((2 * I, CHUNK), jnp.int32),
            pltpu.VMEM((R, CHUNK, D), jnp.float32),
            pltpu.VMEM_SHARED((NPAD, D), jnp.float32),
        ] + [pltpu.SemaphoreType.DMA] * (2 * R + I),
    )
    def sc_agg(x_hbm, sd_hbm, out_hbm, sd_v, rows_v, agg_sh, *sems):
        cid = lax.axis_index("c")
        sid = lax.axis_index("s")
        wid = sid * NC + cid
        sg = sems[:R]
        ss = sems[R:2 * R]
        si = sems[2 * R:]
        zv = jnp.zeros((16,), jnp.float32)

        def zbody(r, c):
            for cc in range(8):
                rows_v[0, r, pl.ds(cc * 16, 16)] = zv
            return c

        lax.fori_loop(0, CHUNK, zbody, 0)
        for t in range(RPT // CHUNK):
            pltpu.sync_copy(rows_v.at[0],
                            agg_sh.at[pl.ds(sid * RPT + t * CHUNK, CHUNK)])
        REM = RPT - (RPT // CHUNK) * CHUNK
        pltpu.sync_copy(rows_v.at[0, pl.ds(0, REM)],
                        agg_sh.at[pl.ds(sid * RPT + RPT - REM, REM)])
        plsc.subcore_barrier()

        def _idx_load(j, bi):
            pltpu.async_copy(sd_hbm.at[wid, j], sd_v.at[pl.ds(2 * bi, 2)],
                             si[bi])

        def _idx_wait(bi):
            pltpu.make_async_copy(sd_hbm.at[wid, 0], sd_v.at[pl.ds(2 * bi, 2)],
                                  si[bi]).wait()

        def _gather(bi, br):
            pltpu.async_copy(x_hbm.at[sd_v.at[2 * bi]], rows_v.at[br], sg[br])

        def _gather_wait(bi, br):
            pltpu.make_async_copy(x_hbm.at[sd_v.at[2 * bi]], rows_v.at[br],
                                  sg[br]).wait()

        def _scatter(bi, br):
            pltpu.async_copy(rows_v.at[br], agg_sh.at[sd_v.at[2 * bi + 1]],
                             ss[br], add=True)

        def _scatter_wait(bi, br):
            pltpu.make_async_copy(rows_v.at[br], agg_sh.at[sd_v.at[2 * bi + 1]],
                                  ss[br]).wait()

        for p in range(R):
            _idx_load(p, p)
        for p in range(R - 1):
            _idx_wait(p)
            _gather(p, p)

        def body(i, carry):
            j = i * UNR
            for u in range(UNR):
                k = j + u
                br = u % R
                bi = u % I
                _gather_wait(bi, br)
                _scatter(bi, br)
                @pl.when(k >= 1)
                def _():
                    _scatter_wait((u + I - 1) % I, (u + R - 1) % R)
                @pl.when(k + R < CPW)
                def _():
                    _idx_load(k + R, (u + R) % I)
                @pl.when(k + R - 1 < CPW)
                def _():
                    _idx_wait((u + R - 1) % I)
                    _gather((u + R - 1) % I, (u + R - 1) % R)
            return carry

        lax.fori_loop(0, CPW // UNR, body, 0)
        _scatter_wait((CPW - 1) % I, (CPW - 1) % R)
        plsc.subcore_barrier()
        pltpu.sync_copy(agg_sh.at[pl.ds(sid * RPT, RPT)],
                        out_hbm.at[cid, pl.ds(sid * RPT, RPT)])

    return sc_agg


_sc_agg = _make_sc_agg()


def _proj_body(a_ref, w1_ref, b1_ref, w2_ref, b2_ref, o_ref):
    t = jnp.dot(a_ref[0] + a_ref[1], w1_ref[...],
                preferred_element_type=jnp.float32)
    h = jnp.maximum(t + b1_ref[...], 0.0)
    o_ref[...] = jnp.sum(h * w2_ref[...], axis=1, keepdims=True) + b2_ref[0, 0]


def _proj(agg2, W1, b1r, w2r, b2r):
    BLK = 2528
    return pl.pallas_call(
        _proj_body,
        grid=(NPAD // BLK,),
        in_specs=[pl.BlockSpec((NC, BLK, D), lambda i: (0, i, 0)),
                  pl.BlockSpec((D, D), lambda i: (0, 0)),
                  pl.BlockSpec((1, D), lambda i: (0, 0)),
                  pl.BlockSpec((1, D), lambda i: (0, 0)),
                  pl.BlockSpec((1, 1), lambda i: (0, 0))],
        out_specs=pl.BlockSpec((BLK, 1), lambda i: (i, 0)),
        out_shape=jax.ShapeDtypeStruct((NPAD, 1), jnp.float32),
    )(agg2, W1, b1r, w2r, b2r)


def kernel(x, edge_index, W1, b1, W2, b2):
    pad = EPAD - E
    pidx = jnp.arange(pad, dtype=jnp.int32)
    pads = jnp.stack([pidx % N, N + pidx % (NPAD - N)])
    pe = jnp.concatenate([edge_index.astype(jnp.int32), pads], axis=1)
    sd4 = pe.reshape(2, NW, CPW, CHUNK).transpose(1, 2, 0, 3)
    agg2 = _sc_agg(x, sd4)
    res = _proj(agg2, W1, b1.reshape(1, D), W2.reshape(1, D), b2.reshape(1, 1))
    return res[:N, 0]
